# baseline (speedup 1.0000x reference)
import functools

import jax
import jax.numpy as jnp
from jax import lax
from jax.experimental import pallas as pl
from jax.experimental.pallas import tpu as pltpu
from jax.experimental.pallas import tpu_sc as plsc

BATCH = 16384
DIM = 32
NC = 2
NS = 16
NW = NC * NS
BPW = BATCH // NW
CHUNK = 128
NCHUNK = BPW // CHUNK


def _sc_gather_body(user_hbm, item_hbm, eug_hbm, eig_hbm, eum_hbm, eim_hbm,
                    out_ug, out_ig, out_um, out_im,
                    uidx_v, iidx_v, ug_v, ig_v, um_v, im_v, sem):
    wid = lax.axis_index("s") * NC + lax.axis_index("c")
    row0 = wid * NCHUNK
    pltpu.sync_copy(user_hbm.at[pl.ds(row0, NCHUNK)], uidx_v)
    pltpu.sync_copy(item_hbm.at[pl.ds(row0, NCHUNK)], iidx_v)
    copies = []
    for j in range(NCHUNK):
        dst = pl.ds(j * CHUNK, CHUNK)
        copies.append(pltpu.async_copy(eug_hbm.at[uidx_v.at[j]], ug_v.at[dst], sem))
        copies.append(pltpu.async_copy(eig_hbm.at[iidx_v.at[j]], ig_v.at[dst], sem))
        copies.append(pltpu.async_copy(eum_hbm.at[uidx_v.at[j]], um_v.at[dst], sem))
        copies.append(pltpu.async_copy(eim_hbm.at[iidx_v.at[j]], im_v.at[dst], sem))
    for c in copies:
        c.wait()
    base = wid * BPW
    pltpu.sync_copy(ug_v, out_ug.at[pl.ds(base, BPW)])
    pltpu.sync_copy(ig_v, out_ig.at[pl.ds(base, BPW)])
    pltpu.sync_copy(um_v, out_um.at[pl.ds(base, BPW)])
    pltpu.sync_copy(im_v, out_im.at[pl.ds(base, BPW)])


@jax.jit
def _sc_gather(user2d, item2d, eug, eig, eum, eim):
    mesh = plsc.VectorSubcoreMesh(core_axis_name="c", subcore_axis_name="s")
    row = jax.ShapeDtypeStruct((BATCH, DIM), jnp.float32)
    fn = pl.kernel(
        _sc_gather_body,
        mesh=mesh,
        compiler_params=pltpu.CompilerParams(use_tc_tiling_on_sc=False),
        out_type=[row, row, row, row],
        scratch_types=[
            pltpu.VMEM((NCHUNK, CHUNK), jnp.int32),
            pltpu.VMEM((NCHUNK, CHUNK), jnp.int32),
            pltpu.VMEM((BPW, DIM), jnp.float32),
            pltpu.VMEM((BPW, DIM), jnp.float32),
            pltpu.VMEM((BPW, DIM), jnp.float32),
            pltpu.VMEM((BPW, DIM), jnp.float32),
            pltpu.SemaphoreType.DMA,
        ],
    )
    return fn(user2d, item2d, eug, eig, eum, eim)


def _tc_body(ug, ig, um, im, W1, b1, W2, b2, W3, b3, wpg, wpm, bp, out):
    x = jnp.concatenate([um[...], im[...]], axis=1)
    h = jnp.maximum(jnp.dot(x, W1[...], preferred_element_type=jnp.float32) + b1[...], 0.0)
    h = jnp.maximum(jnp.dot(h, W2[...], preferred_element_type=jnp.float32) + b2[...], 0.0)
    h = jnp.maximum(jnp.dot(h, W3[...], preferred_element_type=jnp.float32) + b3[...], 0.0)
    gmf = ug[...] * ig[...]
    pred = (jnp.sum(gmf * wpg[...], axis=1) + jnp.sum(h * wpm[...], axis=1)
            + bp[0, 0])
    out[...] = pred


def _tc_call(ug, ig, um, im, W1, b1, W2, b2, W3, b3, wpg, wpm, bp):
    nblk = 8
    blk = BATCH // nblk
    row_spec = pl.BlockSpec((blk, DIM), lambda i: (i, 0))

    def whole(a):
        return pl.BlockSpec(a.shape, lambda i: (0,) * a.ndim)

    return pl.pallas_call(
        _tc_body,
        grid=(nblk,),
        in_specs=[row_spec, row_spec, row_spec, row_spec,
                  whole(W1), whole(b1), whole(W2), whole(b2),
                  whole(W3), whole(b3), whole(wpg), whole(wpm), whole(bp)],
        out_specs=pl.BlockSpec((blk,), lambda i: (i,)),
        out_shape=jax.ShapeDtypeStruct((BATCH,), jnp.float32),
    )(ug, ig, um, im, W1, b1, W2, b2, W3, b3, wpg, wpm, bp)


def kernel(user, item, eu_gmf, ei_gmf, eu_mlp, ei_mlp,
           W1, b1, W2, b2, W3, b3, Wp, bp):
    user2d = user.astype(jnp.int32).reshape(BATCH // CHUNK, CHUNK)
    item2d = item.astype(jnp.int32).reshape(BATCH // CHUNK, CHUNK)
    ug, ig, um, im = _sc_gather(user2d, item2d, eu_gmf, ei_gmf, eu_mlp, ei_mlp)
    wpg = Wp[:DIM].reshape(1, DIM)
    wpm = Wp[DIM:].reshape(1, 16)
    return _tc_call(ug, ig, um, im,
                    W1, b1.reshape(1, -1), W2, b2.reshape(1, -1),
                    W3, b3.reshape(1, -1), wpg, wpm, bp.reshape(1, 1))
